# baseline (speedup 1.0000x reference)
import functools

import jax
import jax.numpy as jnp
from jax.experimental import pallas as pl

D = 2048
E = 8
K = 2
R3 = 512
R2 = 512
B = 4096
EPS = 1e-5
SCALE = 10.0
TEMP = 0.5

T = 512


def _moe_body(x_ref, nw_ref, wr_ref, u_ref, g_ref, v_ref, o_ref):
    x = x_ref[...]
    var = jnp.mean(x * x, axis=-1, keepdims=True)
    xn = x * jax.lax.rsqrt(var + EPS) * nw_ref[...]

    logits = jnp.dot(xn, wr_ref[...], preferred_element_type=jnp.float32)
    col = jax.lax.broadcasted_iota(jnp.int32, (T, E), 1)
    m1 = jnp.max(logits, axis=-1, keepdims=True)
    i1 = jnp.min(jnp.where(logits == m1, col, E), axis=-1, keepdims=True)
    masked = jnp.where(col == i1, -jnp.inf, logits)
    m2 = jnp.max(masked, axis=-1, keepdims=True)
    i2 = jnp.min(jnp.where(masked == m2, col, E), axis=-1, keepdims=True)
    bb = jnp.exp((m2 - m1) / TEMP)
    p1 = 1.0 / (1.0 + bb)
    p2 = 1.0 - p1
    w = jnp.where(col == i1, p1, 0.0) + jnp.where(col == i2, p2, 0.0)

    xs = jnp.tanh(jnp.dot(xn.astype(jnp.bfloat16), u_ref[...],
                          preferred_element_type=jnp.float32)
                  * (1.0 / SCALE)) * SCALE
    xsb = xs.astype(jnp.bfloat16)

    acc = jnp.zeros((T, R2), dtype=jnp.float32)
    for e in range(E):
        he = jnp.dot(xsb, g_ref[e], preferred_element_type=jnp.float32)
        acc = acc + w[:, e:e + 1] * he

    o_ref[...] = jnp.dot(acc.astype(jnp.bfloat16), v_ref[...],
                         preferred_element_type=jnp.float32)


@jax.jit
def kernel(x, norm_w, W_router, U, G, V):
    grid = (B // T,)
    return pl.pallas_call(
        _moe_body,
        grid=grid,
        in_specs=[
            pl.BlockSpec((T, D), lambda i: (i, 0)),
            pl.BlockSpec((1, D), lambda i: (0, 0)),
            pl.BlockSpec((D, E), lambda i: (0, 0)),
            pl.BlockSpec((D, R3), lambda i: (0, 0)),
            pl.BlockSpec((E, R3, R2), lambda i: (0, 0, 0)),
            pl.BlockSpec((R2, D), lambda i: (0, 0)),
        ],
        out_specs=pl.BlockSpec((T, D), lambda i: (i, 0)),
        out_shape=jax.ShapeDtypeStruct((B, D), jnp.float32),
    )(x, norm_w.reshape(1, D), W_router,
      U.astype(jnp.bfloat16), G.astype(jnp.bfloat16), V.astype(jnp.bfloat16))


# speedup vs baseline: 1.0659x; 1.0659x over previous
import jax
import jax.numpy as jnp
from jax.experimental import pallas as pl

D = 2048
E = 8
K = 2
R3 = 512
R2 = 512
B = 4096
EPS = 1e-5
SCALE = 10.0
TEMP = 0.5

T = 512


def _moe_body(x_ref, wr_ref, u_ref, g_ref, v_ref, ones_ref, o_ref):
    x = x_ref[...]
    ssum = jnp.dot(x * x, ones_ref[...], preferred_element_type=jnp.float32)
    s = jax.lax.rsqrt(ssum[:, 0:1] * (1.0 / D) + EPS)

    logits = jnp.dot(x, wr_ref[...], preferred_element_type=jnp.float32) * s
    col = jax.lax.broadcasted_iota(jnp.int32, (T, E), 1)
    m1 = jnp.max(logits, axis=-1, keepdims=True)
    i1 = jnp.min(jnp.where(logits == m1, col, E), axis=-1, keepdims=True)
    masked = jnp.where(col == i1, -jnp.inf, logits)
    m2 = jnp.max(masked, axis=-1, keepdims=True)
    i2 = jnp.min(jnp.where(masked == m2, col, E), axis=-1, keepdims=True)
    bb = jnp.exp((m2 - m1) / TEMP)
    p1 = 1.0 / (1.0 + bb)
    p2 = 1.0 - p1
    w = jnp.where(col == i1, p1, 0.0) + jnp.where(col == i2, p2, 0.0)

    r = jnp.dot(x, u_ref[...], preferred_element_type=jnp.float32)
    xs = jnp.tanh(r * (s * (1.0 / SCALE))) * SCALE

    acc = jnp.zeros((T, R2), dtype=jnp.float32)
    for e in range(E):
        he = jnp.dot(xs, g_ref[e], preferred_element_type=jnp.float32)
        acc = acc + w[:, e:e + 1] * he

    o_ref[...] = jnp.dot(acc, v_ref[...], preferred_element_type=jnp.float32)


@jax.jit
def kernel(x, norm_w, W_router, U, G, V):
    wr2 = W_router * norm_w[:, None]
    u2 = U * norm_w[:, None]
    ones = jnp.ones((D, E), dtype=jnp.float32)
    grid = (B // T,)
    return pl.pallas_call(
        _moe_body,
        grid=grid,
        in_specs=[
            pl.BlockSpec((T, D), lambda i: (i, 0)),
            pl.BlockSpec((D, E), lambda i: (0, 0)),
            pl.BlockSpec((D, R3), lambda i: (0, 0)),
            pl.BlockSpec((E, R3, R2), lambda i: (0, 0, 0)),
            pl.BlockSpec((R2, D), lambda i: (0, 0)),
            pl.BlockSpec((D, E), lambda i: (0, 0)),
        ],
        out_specs=pl.BlockSpec((T, D), lambda i: (i, 0)),
        out_shape=jax.ShapeDtypeStruct((B, D), jnp.float32),
    )(x, wr2, u2, G, V, ones)


# speedup vs baseline: 1.0973x; 1.0295x over previous
import jax
import jax.numpy as jnp
from jax.experimental import pallas as pl

D = 2048
E = 8
K = 2
R3 = 512
R2 = 512
B = 4096
EPS = 1e-5
SCALE = 10.0
TEMP = 0.5

T = 512


def _moe_body(x_ref, nw_ref, wr_ref, u_ref, g_ref, v_ref, o_ref):
    x = x_ref[...]
    var = jnp.mean(x * x, axis=-1, keepdims=True)
    xn = x * jax.lax.rsqrt(var + EPS) * nw_ref[...]

    logits = jnp.dot(xn, wr_ref[...], preferred_element_type=jnp.float32)
    col = jax.lax.broadcasted_iota(jnp.int32, (T, E), 1)
    m1 = jnp.max(logits, axis=-1, keepdims=True)
    i1 = jnp.min(jnp.where(logits == m1, col, E), axis=-1, keepdims=True)
    masked = jnp.where(col == i1, -jnp.inf, logits)
    m2 = jnp.max(masked, axis=-1, keepdims=True)
    i2 = jnp.min(jnp.where(masked == m2, col, E), axis=-1, keepdims=True)
    bb = jnp.exp((m2 - m1) / TEMP)
    p1 = 1.0 / (1.0 + bb)
    p2 = 1.0 - p1
    w = jnp.where(col == i1, p1, 0.0) + jnp.where(col == i2, p2, 0.0)

    xs = jnp.tanh(jnp.dot(xn, u_ref[...], preferred_element_type=jnp.float32)
                  * (1.0 / SCALE)) * SCALE

    acc = jnp.zeros((T, R2), dtype=jnp.float32)
    for e in range(E):
        he = jnp.dot(xs, g_ref[e], preferred_element_type=jnp.float32)
        acc = acc + w[:, e:e + 1] * he

    o_ref[...] = jnp.dot(acc, v_ref[...], preferred_element_type=jnp.float32)


@jax.jit
def kernel(x, norm_w, W_router, U, G, V):
    grid = (B // T,)
    return pl.pallas_call(
        _moe_body,
        grid=grid,
        in_specs=[
            pl.BlockSpec((T, D), lambda i: (i, 0)),
            pl.BlockSpec((1, D), lambda i: (0, 0)),
            pl.BlockSpec((D, E), lambda i: (0, 0)),
            pl.BlockSpec((D, R3), lambda i: (0, 0)),
            pl.BlockSpec((E, R3, R2), lambda i: (0, 0, 0)),
            pl.BlockSpec((R2, D), lambda i: (0, 0)),
        ],
        out_specs=pl.BlockSpec((T, D), lambda i: (i, 0)),
        out_shape=jax.ShapeDtypeStruct((B, D), jnp.float32),
    )(x, norm_w.reshape(1, D), W_router, U, G, V)
